# baseline (speedup 1.0000x reference)
import functools

import jax
import jax.numpy as jnp
from jax import lax
from jax.experimental import pallas as pl
from jax.experimental.pallas import tpu as pltpu
from jax.experimental.pallas import tpu_sc as plsc

VOCAB = 30522
MAX_POS = 512
HIDDEN = 768
EPS = 1e-12

NC = 2
NS = 16
NW = NC * NS
CHUNK = 64


def _sc_word_gather(word_w, idx1, tok0, n_tokens):
    b_per_w = n_tokens // NW
    n_chunks = b_per_w // CHUNK
    mesh = plsc.VectorSubcoreMesh(core_axis_name="c", subcore_axis_name="s")

    @functools.partial(
        pl.kernel,
        mesh=mesh,
        out_type=jax.ShapeDtypeStruct((n_tokens, HIDDEN), jnp.float32),
        scratch_types=[
            pltpu.VMEM((b_per_w,), jnp.int32),
            pltpu.VMEM((2, CHUNK, HIDDEN), jnp.float32),
            pltpu.SemaphoreType.DMA,
            pltpu.SemaphoreType.DMA,
            pltpu.SemaphoreType.DMA,
        ],
    )
    def gather_kernel(table_hbm, idx_hbm, out_hbm, idx_v, rows_v, sem_g,
                      sem_w0, sem_w1):
        wid = lax.axis_index("s") * NC + lax.axis_index("c")
        base = wid * b_per_w
        pltpu.sync_copy(idx_hbm.at[pl.ds(tok0 + base, b_per_w)], idx_v)
        sem_w = (sem_w0, sem_w1)

        def gather_to(j, buf):
            return pltpu.make_async_copy(
                table_hbm.at[idx_v.at[pl.ds(j * CHUNK, CHUNK)]],
                rows_v.at[buf], sem_g)

        def write_from(j, buf):
            return pltpu.make_async_copy(
                rows_v.at[buf], out_hbm.at[pl.ds(base + j * CHUNK, CHUNK)],
                sem_w[buf])

        gather_to(0, 0).start()
        for j in range(n_chunks):
            b = j & 1
            gather_to(j, b).wait()
            if j + 1 < n_chunks:
                if j >= 1:
                    write_from(j - 1, 1 - b).wait()
                gather_to(j + 1, 1 - b).start()
            write_from(j, b).start()
        write_from(n_chunks - 1, (n_chunks - 1) & 1).wait()
        if n_chunks >= 2:
            write_from(n_chunks - 2, (n_chunks - 2) & 1).wait()

    return gather_kernel(word_w, idx1)


def _tc_fuse_kernel(w_ref, pid_ref, tid_ref, posw_ref, typew_ref, lnw_ref,
                    lnb_ref, out_ref):
    pid = pid_ref[0, 0, :]
    tid = tid_ref[0, 0, :]
    onehot = (pid[:, None] == lax.broadcasted_iota(
        jnp.int32, (pid.shape[0], MAX_POS), 1)).astype(jnp.bfloat16)
    p = lax.dot_general(
        onehot, posw_ref[...].astype(jnp.bfloat16),
        (((1,), (0,)), ((), ())), preferred_element_type=jnp.float32)
    row0 = typew_ref[0:1, :]
    row1 = typew_ref[1:2, :]
    t = tid.astype(jnp.float32)[:, None]
    x = w_ref[...] + p + row0 + t * (row1 - row0)
    mean = jnp.mean(x, axis=1, keepdims=True)
    xc = x - mean
    var = jnp.mean(xc * xc, axis=1, keepdims=True)
    y = xc * lax.rsqrt(var + EPS)
    out_ref[...] = y * lnw_ref[...] + lnb_ref[...]


def _tc_fuse_alias_kernel(prev_ref, w_ref, pid_ref, tid_ref, posw_ref,
                          typew_ref, lnw_ref, lnb_ref, out_ref):
    del prev_ref
    _tc_fuse_kernel(w_ref, pid_ref, tid_ref, posw_ref, typew_ref, lnw_ref,
                    lnb_ref, out_ref)


SLICE_BLOCKS = (4, 12, 12, 4)
BLOCK_T = 512


def kernel(input_ids, token_type_ids, position_ids, word_w, pos_w, type_w,
           ln_w, ln_b):
    batch, seq = input_ids.shape
    n_tokens = batch * seq
    n_blocks = n_tokens // BLOCK_T
    assert sum(SLICE_BLOCKS) == n_blocks

    ids1 = input_ids.astype(jnp.int32).reshape(n_tokens)
    w_slices = []
    off = 0
    for bl in SLICE_BLOCKS:
        w_slices.append(_sc_word_gather(word_w, ids1, off * BLOCK_T,
                                        bl * BLOCK_T))
        off += bl

    pid3 = position_ids.astype(jnp.int32).reshape(n_blocks, 1, BLOCK_T)
    tid3 = token_type_ids.astype(jnp.int32).reshape(n_blocks, 1, BLOCK_T)
    lnw2 = ln_w.reshape(1, HIDDEN)
    lnb2 = ln_b.reshape(1, HIDDEN)

    out = None
    off = 0
    for s, bl in enumerate(SLICE_BLOCKS):
        in_specs = [
            pl.BlockSpec((BLOCK_T, HIDDEN), lambda i: (i, 0)),
            pl.BlockSpec((1, 1, BLOCK_T), lambda i, o=off: (i + o, 0, 0)),
            pl.BlockSpec((1, 1, BLOCK_T), lambda i, o=off: (i + o, 0, 0)),
            pl.BlockSpec((MAX_POS, HIDDEN), lambda i: (0, 0)),
            pl.BlockSpec((2, HIDDEN), lambda i: (0, 0)),
            pl.BlockSpec((1, HIDDEN), lambda i: (0, 0)),
            pl.BlockSpec((1, HIDDEN), lambda i: (0, 0)),
        ]
        out_spec = pl.BlockSpec((BLOCK_T, HIDDEN), lambda i, o=off: (i + o, 0))
        args = (w_slices[s], pid3, tid3, pos_w, type_w, lnw2, lnb2)
        if s == 0:
            out = pl.pallas_call(
                _tc_fuse_kernel,
                grid=(bl,),
                in_specs=in_specs,
                out_specs=out_spec,
                out_shape=jax.ShapeDtypeStruct((n_tokens, HIDDEN), jnp.float32),
            )(*args)
        else:
            out = pl.pallas_call(
                _tc_fuse_alias_kernel,
                grid=(bl,),
                in_specs=[pl.BlockSpec(memory_space=pl.ANY)] + in_specs,
                out_specs=out_spec,
                out_shape=jax.ShapeDtypeStruct((n_tokens, HIDDEN), jnp.float32),
                input_output_aliases={0: 0},
            )(out, *args)
        off += bl

    return out.reshape(batch, seq, HIDDEN)


# speedup vs baseline: 1.0080x; 1.0080x over previous
import functools

import jax
import jax.numpy as jnp
from jax import lax
from jax.experimental import pallas as pl
from jax.experimental.pallas import tpu as pltpu
from jax.experimental.pallas import tpu_sc as plsc

VOCAB = 30522
MAX_POS = 512
HIDDEN = 768
EPS = 1e-12

NC = 2
NS = 16
NW = NC * NS
CHUNK = 64


def _sc_word_gather(word_w, idx1, tok0, n_tokens):
    b_per_w = n_tokens // NW
    n_chunks = b_per_w // CHUNK
    mesh = plsc.VectorSubcoreMesh(core_axis_name="c", subcore_axis_name="s")

    @functools.partial(
        pl.kernel,
        mesh=mesh,
        out_type=jax.ShapeDtypeStruct((n_tokens, HIDDEN), jnp.float32),
        scratch_types=[
            pltpu.VMEM((b_per_w,), jnp.int32),
            pltpu.VMEM((2, CHUNK, HIDDEN), jnp.float32),
            pltpu.SemaphoreType.DMA,
            pltpu.SemaphoreType.DMA,
            pltpu.SemaphoreType.DMA,
        ],
    )
    def gather_kernel(table_hbm, idx_hbm, out_hbm, idx_v, rows_v, sem_g,
                      sem_w0, sem_w1):
        wid = lax.axis_index("s") * NC + lax.axis_index("c")
        base = wid * b_per_w
        pltpu.sync_copy(idx_hbm.at[pl.ds(tok0 + base, b_per_w)], idx_v)
        sem_w = (sem_w0, sem_w1)

        def gather_to(j, buf):
            return pltpu.make_async_copy(
                table_hbm.at[idx_v.at[pl.ds(j * CHUNK, CHUNK)]],
                rows_v.at[buf], sem_g)

        def write_from(j, buf):
            return pltpu.make_async_copy(
                rows_v.at[buf], out_hbm.at[pl.ds(base + j * CHUNK, CHUNK)],
                sem_w[buf])

        gather_to(0, 0).start()
        for j in range(n_chunks):
            b = j & 1
            gather_to(j, b).wait()
            if j + 1 < n_chunks:
                if j >= 1:
                    write_from(j - 1, 1 - b).wait()
                gather_to(j + 1, 1 - b).start()
            write_from(j, b).start()
        write_from(n_chunks - 1, (n_chunks - 1) & 1).wait()
        if n_chunks >= 2:
            write_from(n_chunks - 2, (n_chunks - 2) & 1).wait()

    return gather_kernel(word_w, idx1)


def _tc_fuse_kernel(w_ref, pid_ref, tid_ref, posw_ref, typew_ref, lnw_ref,
                    lnb_ref, out_ref):
    pid = pid_ref[0, 0, :]
    tid = tid_ref[0, 0, :]
    onehot = (pid[:, None] == lax.broadcasted_iota(
        jnp.int32, (pid.shape[0], MAX_POS), 1)).astype(jnp.bfloat16)
    p = lax.dot_general(
        onehot, posw_ref[...].astype(jnp.bfloat16),
        (((1,), (0,)), ((), ())), preferred_element_type=jnp.float32)
    row0 = typew_ref[0:1, :]
    row1 = typew_ref[1:2, :]
    t = tid.astype(jnp.float32)[:, None]
    x = w_ref[...] + p + row0 + t * (row1 - row0)
    mean = jnp.mean(x, axis=1, keepdims=True)
    xc = x - mean
    var = jnp.mean(xc * xc, axis=1, keepdims=True)
    y = xc * lax.rsqrt(var + EPS)
    out_ref[...] = y * lnw_ref[...] + lnb_ref[...]


def _tc_fuse_alias_kernel(prev_ref, w_ref, pid_ref, tid_ref, posw_ref,
                          typew_ref, lnw_ref, lnb_ref, out_ref):
    del prev_ref
    _tc_fuse_kernel(w_ref, pid_ref, tid_ref, posw_ref, typew_ref, lnw_ref,
                    lnb_ref, out_ref)


SLICE_BLOCKS = (16, 16)
BLOCK_T = 512


def kernel(input_ids, token_type_ids, position_ids, word_w, pos_w, type_w,
           ln_w, ln_b):
    batch, seq = input_ids.shape
    n_tokens = batch * seq
    n_blocks = n_tokens // BLOCK_T
    assert sum(SLICE_BLOCKS) == n_blocks

    ids1 = input_ids.astype(jnp.int32).reshape(n_tokens)
    w_slices = []
    off = 0
    for bl in SLICE_BLOCKS:
        w_slices.append(_sc_word_gather(word_w, ids1, off * BLOCK_T,
                                        bl * BLOCK_T))
        off += bl

    pid3 = position_ids.astype(jnp.int32).reshape(n_blocks, 1, BLOCK_T)
    tid3 = token_type_ids.astype(jnp.int32).reshape(n_blocks, 1, BLOCK_T)
    lnw2 = ln_w.reshape(1, HIDDEN)
    lnb2 = ln_b.reshape(1, HIDDEN)

    out = None
    off = 0
    for s, bl in enumerate(SLICE_BLOCKS):
        in_specs = [
            pl.BlockSpec((BLOCK_T, HIDDEN), lambda i: (i, 0)),
            pl.BlockSpec((1, 1, BLOCK_T), lambda i, o=off: (i + o, 0, 0)),
            pl.BlockSpec((1, 1, BLOCK_T), lambda i, o=off: (i + o, 0, 0)),
            pl.BlockSpec((MAX_POS, HIDDEN), lambda i: (0, 0)),
            pl.BlockSpec((2, HIDDEN), lambda i: (0, 0)),
            pl.BlockSpec((1, HIDDEN), lambda i: (0, 0)),
            pl.BlockSpec((1, HIDDEN), lambda i: (0, 0)),
        ]
        out_spec = pl.BlockSpec((BLOCK_T, HIDDEN), lambda i, o=off: (i + o, 0))
        args = (w_slices[s], pid3, tid3, pos_w, type_w, lnw2, lnb2)
        if s == 0:
            out = pl.pallas_call(
                _tc_fuse_kernel,
                grid=(bl,),
                in_specs=in_specs,
                out_specs=out_spec,
                out_shape=jax.ShapeDtypeStruct((n_tokens, HIDDEN), jnp.float32),
            )(*args)
        else:
            out = pl.pallas_call(
                _tc_fuse_alias_kernel,
                grid=(bl,),
                in_specs=[pl.BlockSpec(memory_space=pl.ANY)] + in_specs,
                out_specs=out_spec,
                out_shape=jax.ShapeDtypeStruct((n_tokens, HIDDEN), jnp.float32),
                input_output_aliases={0: 0},
            )(out, *args)
        off += bl

    return out.reshape(batch, seq, HIDDEN)


# speedup vs baseline: 1.0379x; 1.0296x over previous
import functools

import jax
import jax.numpy as jnp
from jax import lax
from jax.experimental import pallas as pl
from jax.experimental.pallas import tpu as pltpu
from jax.experimental.pallas import tpu_sc as plsc

VOCAB = 30522
MAX_POS = 512
HIDDEN = 768
EPS = 1e-12

NC = 2
NS = 16
NW = NC * NS
CHUNK = 64


def _sc_word_gather(word_w, idx1, tok0, n_tokens):
    b_per_w = n_tokens // NW
    n_chunks = b_per_w // CHUNK
    mesh = plsc.VectorSubcoreMesh(core_axis_name="c", subcore_axis_name="s")

    @functools.partial(
        pl.kernel,
        mesh=mesh,
        out_type=jax.ShapeDtypeStruct((n_tokens, HIDDEN), jnp.float32),
        scratch_types=[
            pltpu.VMEM((b_per_w,), jnp.int32),
            pltpu.VMEM((2, CHUNK, HIDDEN), jnp.float32),
            pltpu.SemaphoreType.DMA,
            pltpu.SemaphoreType.DMA,
            pltpu.SemaphoreType.DMA,
        ],
    )
    def gather_kernel(table_hbm, idx_hbm, out_hbm, idx_v, rows_v, sem_g,
                      sem_w0, sem_w1):
        wid = lax.axis_index("s") * NC + lax.axis_index("c")
        base = wid * b_per_w
        pltpu.sync_copy(idx_hbm.at[pl.ds(tok0 + base, b_per_w)], idx_v)
        sem_w = (sem_w0, sem_w1)

        def gather_to(j, buf):
            return pltpu.make_async_copy(
                table_hbm.at[idx_v.at[pl.ds(j * CHUNK, CHUNK)]],
                rows_v.at[buf], sem_g)

        def write_from(j, buf):
            return pltpu.make_async_copy(
                rows_v.at[buf], out_hbm.at[pl.ds(base + j * CHUNK, CHUNK)],
                sem_w[buf])

        gather_to(0, 0).start()
        for j in range(n_chunks):
            b = j & 1
            gather_to(j, b).wait()
            if j + 1 < n_chunks:
                if j >= 1:
                    write_from(j - 1, 1 - b).wait()
                gather_to(j + 1, 1 - b).start()
            write_from(j, b).start()
        write_from(n_chunks - 1, (n_chunks - 1) & 1).wait()
        if n_chunks >= 2:
            write_from(n_chunks - 2, (n_chunks - 2) & 1).wait()

    return gather_kernel(word_w, idx1)


def _tc_fuse_kernel(w_ref, pid_ref, tid_ref, posw_ref, typew_ref, lnw_ref,
                    lnb_ref, out_ref):
    pid = pid_ref[0, 0, :]
    tid = tid_ref[0, 0, :]
    onehot = (pid[:, None] == lax.broadcasted_iota(
        jnp.int32, (pid.shape[0], MAX_POS), 1)).astype(jnp.bfloat16)
    p = lax.dot_general(
        onehot, posw_ref[...],
        (((1,), (0,)), ((), ())), preferred_element_type=jnp.float32)
    row0 = typew_ref[0:1, :]
    row1 = typew_ref[1:2, :]
    t = tid.astype(jnp.float32)[:, None]
    x = w_ref[...] + p + row0 + t * (row1 - row0)
    mean = jnp.mean(x, axis=1, keepdims=True)
    xc = x - mean
    var = jnp.mean(xc * xc, axis=1, keepdims=True)
    y = xc * lax.rsqrt(var + EPS)
    out_ref[...] = y * lnw_ref[...] + lnb_ref[...]


def _tc_fuse_alias_kernel(prev_ref, w_ref, pid_ref, tid_ref, posw_ref,
                          typew_ref, lnw_ref, lnb_ref, out_ref):
    del prev_ref
    _tc_fuse_kernel(w_ref, pid_ref, tid_ref, posw_ref, typew_ref, lnw_ref,
                    lnb_ref, out_ref)


SLICE_BLOCKS = (4, 4, 4, 4)
BLOCK_T = 1024


def kernel(input_ids, token_type_ids, position_ids, word_w, pos_w, type_w,
           ln_w, ln_b):
    batch, seq = input_ids.shape
    n_tokens = batch * seq
    n_blocks = n_tokens // BLOCK_T
    assert sum(SLICE_BLOCKS) == n_blocks

    ids1 = input_ids.astype(jnp.int32).reshape(n_tokens)
    w_slices = []
    off = 0
    for bl in SLICE_BLOCKS:
        w_slices.append(_sc_word_gather(word_w, ids1, off * BLOCK_T,
                                        bl * BLOCK_T))
        off += bl

    pid3 = position_ids.astype(jnp.int32).reshape(n_blocks, 1, BLOCK_T)
    tid3 = token_type_ids.astype(jnp.int32).reshape(n_blocks, 1, BLOCK_T)
    lnw2 = ln_w.reshape(1, HIDDEN)
    lnb2 = ln_b.reshape(1, HIDDEN)
    posw_bf16 = pos_w.astype(jnp.bfloat16)

    out = None
    off = 0
    for s, bl in enumerate(SLICE_BLOCKS):
        in_specs = [
            pl.BlockSpec((BLOCK_T, HIDDEN), lambda i: (i, 0)),
            pl.BlockSpec((1, 1, BLOCK_T), lambda i, o=off: (i + o, 0, 0)),
            pl.BlockSpec((1, 1, BLOCK_T), lambda i, o=off: (i + o, 0, 0)),
            pl.BlockSpec((MAX_POS, HIDDEN), lambda i: (0, 0)),
            pl.BlockSpec((2, HIDDEN), lambda i: (0, 0)),
            pl.BlockSpec((1, HIDDEN), lambda i: (0, 0)),
            pl.BlockSpec((1, HIDDEN), lambda i: (0, 0)),
        ]
        out_spec = pl.BlockSpec((BLOCK_T, HIDDEN), lambda i, o=off: (i + o, 0))
        args = (w_slices[s], pid3, tid3, posw_bf16, type_w, lnw2, lnb2)
        if s == 0:
            out = pl.pallas_call(
                _tc_fuse_kernel,
                grid=(bl,),
                in_specs=in_specs,
                out_specs=out_spec,
                out_shape=jax.ShapeDtypeStruct((n_tokens, HIDDEN), jnp.float32),
            )(*args)
        else:
            out = pl.pallas_call(
                _tc_fuse_alias_kernel,
                grid=(bl,),
                in_specs=[pl.BlockSpec(memory_space=pl.ANY)] + in_specs,
                out_specs=out_spec,
                out_shape=jax.ShapeDtypeStruct((n_tokens, HIDDEN), jnp.float32),
                input_output_aliases={0: 0},
            )(out, *args)
        off += bl

    return out.reshape(batch, seq, HIDDEN)
